# Initial kernel scaffold; baseline (speedup 1.0000x reference)
import jax
import jax.numpy as jnp
from jax.experimental import pallas as pl


def kernel(x, emb_weight):
    raise NotImplementedError("write your pallas kernel here")


# speedup vs baseline: 4.3934x; 4.3934x over previous
import functools

import jax
import jax.numpy as jnp
from jax import lax
from jax.experimental import pallas as pl
from jax.experimental.pallas import tpu as pltpu
from jax.experimental.pallas import tpu_sc as plsc

N_EMB = 8192
DIM = 256
N_ROWS = 8192
RB = 256
WIN = 2048
COMMIT = 0.25


def _argmin_body(x_ref, xsq_ref, esq_ref, emb_ref, idx_ref, loss_ref):
    i = pl.program_id(0)

    x = x_ref[...]
    xsq = xsq_ref[...]
    esq = esq_ref[...]
    e = emb_ref[...]

    dot = lax.dot_general(x, e, (((1,), (1,)), ((), ())),
                          preferred_element_type=jnp.float32)
    d2 = (xsq + esq) - 2.0 * dot
    s = jnp.sqrt(jnp.maximum(d2, 0.0))

    col = lax.broadcasted_iota(jnp.int32, s.shape, 1)
    acc_v = acc_i = acc_t = None
    for w in range(0, N_EMB, WIN):
        hi = min(w + WIN, N_EMB)
        mask = (col >= w) & (col < hi)
        sw = jnp.where(mask, s, jnp.float32(jnp.inf))
        lmin = jnp.min(sw, axis=1, keepdims=True)
        lidx = jnp.min(jnp.where(sw == lmin, col, jnp.int32(2**30)),
                       axis=1, keepdims=True)
        if acc_v is None:
            acc_v, acc_i, acc_t = lmin, lidx, lmin
        else:
            accr = acc_v.astype(jnp.bfloat16).astype(jnp.float32)
            take = lmin < accr
            acc_v = jnp.where(take, lmin, accr)
            acc_i = jnp.where(take, lidx, acc_i)
            acc_t = jnp.where(take, lmin, acc_t)

    idx_ref[...] = acc_i
    part = jnp.sum(acc_t * acc_t)
    prev = jnp.where(i == 0, jnp.float32(0.0), loss_ref[0, 0])
    loss_ref[0, 0] = prev + part


def _argmin_call(flat, xsq, esq, emb):
    return pl.pallas_call(
        _argmin_body,
        grid=(N_ROWS // RB,),
        in_specs=[
            pl.BlockSpec((RB, DIM), lambda i: (i, 0)),
            pl.BlockSpec((RB, 1), lambda i: (i, 0)),
            pl.BlockSpec((1, N_EMB), lambda i: (0, 0)),
            pl.BlockSpec((N_EMB, DIM), lambda i: (0, 0)),
        ],
        out_specs=[
            pl.BlockSpec((RB, 1), lambda i: (i, 0)),
            pl.BlockSpec(memory_space=pltpu.SMEM, block_shape=(1, 1),
                         index_map=lambda i: (0, 0)),
        ],
        out_shape=[
            jax.ShapeDtypeStruct((N_ROWS, 1), jnp.int32),
            jax.ShapeDtypeStruct((1, 1), jnp.float32),
        ],
        compiler_params=pltpu.CompilerParams(
            dimension_semantics=("arbitrary",)),
    )(flat, xsq, esq, emb)


def _build_gather():
    info = plsc.get_sparse_core_info()
    nw = info.num_cores * info.num_subcores
    b_per_w = N_ROWS // nw
    mesh = plsc.VectorSubcoreMesh(core_axis_name="c", subcore_axis_name="s")

    @functools.partial(
        pl.kernel, mesh=mesh,
        out_type=jax.ShapeDtypeStruct((N_ROWS, DIM), jnp.float32),
        scratch_types=[
            pltpu.VMEM((b_per_w,), jnp.int32),
            pltpu.VMEM((b_per_w, DIM), jnp.float32),
            pltpu.SemaphoreType.DMA,
        ],
    )
    def gather_k(emb_hbm, idx_hbm, out_hbm, idx_v, rows_v, sem):
        wid = lax.axis_index("s") * info.num_cores + lax.axis_index("c")
        base = wid * b_per_w
        pltpu.sync_copy(idx_hbm.at[pl.ds(base, b_per_w)], idx_v)
        pltpu.async_copy(emb_hbm.at[idx_v], rows_v, sem).wait()
        pltpu.sync_copy(rows_v, out_hbm.at[pl.ds(base, b_per_w)])

    return gather_k


_cached_gather = functools.cache(_build_gather)


def kernel(x, emb_weight):
    flat = x.reshape(-1, DIM)
    xsq = jnp.sum(flat ** 2, axis=1, keepdims=True)
    esq = jnp.sum(emb_weight ** 2, axis=1).reshape(1, N_EMB)
    idx2, loss_sum = _argmin_call(flat, xsq, esq, emb_weight)
    quant_flat = _cached_gather()(emb_weight, idx2.reshape(-1))
    quantized = quant_flat.reshape(x.shape)
    loss = (1.0 + COMMIT) * loss_sum[0, 0] / jnp.float32(N_ROWS * DIM)
    return quantized, loss, idx2


# speedup vs baseline: 5.7909x; 1.3181x over previous
import functools

import jax
import jax.numpy as jnp
from jax import lax
from jax.experimental import pallas as pl
from jax.experimental.pallas import tpu as pltpu
from jax.experimental.pallas import tpu_sc as plsc

N_EMB = 8192
DIM = 256
N_ROWS = 8192
RB = 256
WIN = 2048
COMMIT = 0.25


def _argmin_body(x_ref, xsq_ref, esq_ref, emb_ref, idx_ref, loss_ref):
    i = pl.program_id(0)

    x = x_ref[...]
    xsq = xsq_ref[...]
    esq = esq_ref[...]
    e = emb_ref[...]

    dot = lax.dot_general(x, e, (((1,), (1,)), ((), ())),
                          preferred_element_type=jnp.float32)
    d2 = (xsq + esq) - 2.0 * dot
    s = jnp.sqrt(jnp.maximum(d2, 0.0))

    col = lax.broadcasted_iota(jnp.int32, (RB, WIN), 1)
    acc_v = acc_i = acc_t = None
    for w in range(0, N_EMB, WIN):
        sw = s[:, w:w + WIN]
        lmin = jnp.min(sw, axis=1, keepdims=True)
        lidx = jnp.min(jnp.where(sw == lmin, col + w, jnp.int32(2**30)),
                       axis=1, keepdims=True)
        if acc_v is None:
            acc_v, acc_i, acc_t = lmin, lidx, lmin
        else:
            accr = acc_v.astype(jnp.bfloat16).astype(jnp.float32)
            take = lmin < accr
            acc_v = jnp.where(take, lmin, accr)
            acc_i = jnp.where(take, lidx, acc_i)
            acc_t = jnp.where(take, lmin, acc_t)

    idx_ref[...] = acc_i
    part = jnp.sum(acc_t * acc_t)
    prev = jnp.where(i == 0, jnp.float32(0.0), loss_ref[0, 0])
    loss_ref[0, 0] = prev + part


def _argmin_call(flat, xsq, esq, emb):
    return pl.pallas_call(
        _argmin_body,
        grid=(N_ROWS // RB,),
        in_specs=[
            pl.BlockSpec((RB, DIM), lambda i: (i, 0)),
            pl.BlockSpec((RB, 1), lambda i: (i, 0)),
            pl.BlockSpec((1, N_EMB), lambda i: (0, 0)),
            pl.BlockSpec((N_EMB, DIM), lambda i: (0, 0)),
        ],
        out_specs=[
            pl.BlockSpec((RB, 1), lambda i: (i, 0)),
            pl.BlockSpec(memory_space=pltpu.SMEM, block_shape=(1, 1),
                         index_map=lambda i: (0, 0)),
        ],
        out_shape=[
            jax.ShapeDtypeStruct((N_ROWS, 1), jnp.int32),
            jax.ShapeDtypeStruct((1, 1), jnp.float32),
        ],
        compiler_params=pltpu.CompilerParams(
            dimension_semantics=("arbitrary",)),
    )(flat, xsq, esq, emb)


def _build_gather():
    info = plsc.get_sparse_core_info()
    nw = info.num_cores * info.num_subcores
    b_per_w = N_ROWS // nw
    mesh = plsc.VectorSubcoreMesh(core_axis_name="c", subcore_axis_name="s")

    @functools.partial(
        pl.kernel, mesh=mesh,
        out_type=jax.ShapeDtypeStruct((N_ROWS, DIM), jnp.float32),
        scratch_types=[
            pltpu.VMEM((b_per_w,), jnp.int32),
            pltpu.VMEM((b_per_w, DIM), jnp.float32),
            pltpu.SemaphoreType.DMA,
        ],
    )
    def gather_k(emb_hbm, idx_hbm, out_hbm, idx_v, rows_v, sem):
        wid = lax.axis_index("s") * info.num_cores + lax.axis_index("c")
        base = wid * b_per_w
        pltpu.sync_copy(idx_hbm.at[pl.ds(base, b_per_w)], idx_v)
        pltpu.async_copy(emb_hbm.at[idx_v], rows_v, sem).wait()
        pltpu.sync_copy(rows_v, out_hbm.at[pl.ds(base, b_per_w)])

    return gather_k


_cached_gather = functools.cache(_build_gather)


def kernel(x, emb_weight):
    flat = x.reshape(-1, DIM)
    xsq = jnp.sum(flat ** 2, axis=1, keepdims=True)
    esq = jnp.sum(emb_weight ** 2, axis=1).reshape(1, N_EMB)
    idx2, loss_sum = _argmin_call(flat, xsq, esq, emb_weight)
    quant_flat = _cached_gather()(emb_weight, idx2.reshape(-1))
    quantized = quant_flat.reshape(x.shape)
    loss = (1.0 + COMMIT) * loss_sum[0, 0] / jnp.float32(N_ROWS * DIM)
    return quantized, loss, idx2


# speedup vs baseline: 6.3025x; 1.0883x over previous
import functools

import jax
import jax.numpy as jnp
from jax import lax
from jax.experimental import pallas as pl
from jax.experimental.pallas import tpu as pltpu
from jax.experimental.pallas import tpu_sc as plsc

N_EMB = 8192
DIM = 256
N_ROWS = 8192
RB = 512
WIN = 2048
COMMIT = 0.25


def _argmin_body(x_ref, xsq_ref, esq_ref, emb_ref, idx_ref, loss_ref):
    i = pl.program_id(0)

    x = x_ref[...]
    xsq = xsq_ref[...]
    esq = esq_ref[...]
    e = emb_ref[...]

    dot = lax.dot_general(x, e, (((1,), (1,)), ((), ())),
                          preferred_element_type=jnp.float32)
    d2 = (xsq + esq) - 2.0 * dot
    s = jnp.sqrt(jnp.maximum(d2, 0.0))

    col = lax.broadcasted_iota(jnp.int32, (RB, WIN), 1)
    acc_v = acc_i = acc_t = None
    for w in range(0, N_EMB, WIN):
        sw = s[:, w:w + WIN]
        lmin = jnp.min(sw, axis=1, keepdims=True)
        lidx = jnp.min(jnp.where(sw == lmin, col + w, jnp.int32(2**30)),
                       axis=1, keepdims=True)
        if acc_v is None:
            acc_v, acc_i, acc_t = lmin, lidx, lmin
        else:
            accr = acc_v.astype(jnp.bfloat16).astype(jnp.float32)
            take = lmin < accr
            acc_v = jnp.where(take, lmin, accr)
            acc_i = jnp.where(take, lidx, acc_i)
            acc_t = jnp.where(take, lmin, acc_t)

    idx_ref[...] = acc_i
    part = jnp.sum(acc_t * acc_t)
    prev = jnp.where(i == 0, jnp.float32(0.0), loss_ref[0, 0])
    loss_ref[0, 0] = prev + part


def _argmin_call(flat, xsq, esq, emb):
    return pl.pallas_call(
        _argmin_body,
        grid=(N_ROWS // RB,),
        in_specs=[
            pl.BlockSpec((RB, DIM), lambda i: (i, 0)),
            pl.BlockSpec((RB, 1), lambda i: (i, 0)),
            pl.BlockSpec((1, N_EMB), lambda i: (0, 0)),
            pl.BlockSpec((N_EMB, DIM), lambda i: (0, 0)),
        ],
        out_specs=[
            pl.BlockSpec((RB, 1), lambda i: (i, 0)),
            pl.BlockSpec(memory_space=pltpu.SMEM, block_shape=(1, 1),
                         index_map=lambda i: (0, 0)),
        ],
        out_shape=[
            jax.ShapeDtypeStruct((N_ROWS, 1), jnp.int32),
            jax.ShapeDtypeStruct((1, 1), jnp.float32),
        ],
        compiler_params=pltpu.CompilerParams(
            dimension_semantics=("arbitrary",)),
    )(flat, xsq, esq, emb)


def _build_gather():
    info = plsc.get_sparse_core_info()
    nw = info.num_cores * info.num_subcores
    b_per_w = N_ROWS // nw
    mesh = plsc.VectorSubcoreMesh(core_axis_name="c", subcore_axis_name="s")

    @functools.partial(
        pl.kernel, mesh=mesh,
        out_type=jax.ShapeDtypeStruct((N_ROWS, DIM), jnp.float32),
        scratch_types=[
            pltpu.VMEM((b_per_w,), jnp.int32),
            pltpu.VMEM((b_per_w, DIM), jnp.float32),
            pltpu.SemaphoreType.DMA,
        ],
    )
    def gather_k(emb_hbm, idx_hbm, out_hbm, idx_v, rows_v, sem):
        wid = lax.axis_index("s") * info.num_cores + lax.axis_index("c")
        base = wid * b_per_w
        pltpu.sync_copy(idx_hbm.at[pl.ds(base, b_per_w)], idx_v)
        pltpu.async_copy(emb_hbm.at[idx_v], rows_v, sem).wait()
        pltpu.sync_copy(rows_v, out_hbm.at[pl.ds(base, b_per_w)])

    return gather_k


_cached_gather = functools.cache(_build_gather)


def kernel(x, emb_weight):
    flat = x.reshape(-1, DIM)
    xsq = jnp.sum(flat ** 2, axis=1, keepdims=True)
    esq = jnp.sum(emb_weight ** 2, axis=1).reshape(1, N_EMB)
    idx2, loss_sum = _argmin_call(flat, xsq, esq, emb_weight)
    quant_flat = _cached_gather()(emb_weight, idx2.reshape(-1))
    quantized = quant_flat.reshape(x.shape)
    loss = (1.0 + COMMIT) * loss_sum[0, 0] / jnp.float32(N_ROWS * DIM)
    return quantized, loss, idx2
